# baseline (speedup 1.0000x reference)
import functools

import jax
import jax.numpy as jnp
import numpy as np
from jax.experimental import pallas as pl

_I32_SIGN = np.int32(-2147483648)


def _conf_kernel(pred_t_ref, gt_ref, conf_ref):
    x = pred_t_ref[0]
    m = jnp.max(x, axis=0, keepdims=True)
    e = jnp.exp(x - m)
    s = jnp.sum(e, axis=0, keepdims=True)
    lse = m + jnp.log(s)
    gt = gt_ref[0]
    cio = jax.lax.broadcasted_iota(jnp.int32, x.shape, 0)
    xg = jnp.sum(jnp.where(cio == gt, x, 0.0), axis=0, keepdims=True)
    conf_ref[0] = lse - xg


def _box_kernel(gb_ref, pb_ref, box_ref):
    step = pl.program_id(0)

    @pl.when(step == 0)
    def _init():
        box_ref[...] = jnp.zeros_like(box_ref)

    d = pb_ref[...] - gb_ref[...]
    ad = jnp.abs(d)
    part = jnp.sum(jnp.where(ad < 1.0, 0.5 * d * d, ad - 0.5))
    box_ref[...] = box_ref[...] + jnp.reshape(part, (1, 1))


def _loss_kernel(conf_ref, gt_ref, box_ref,
                 reg_ref, cls_ref, cls_acc, np_acc,
                 *, nsteps, n, ratio):
    step = pl.program_id(0)

    @pl.when(step == 0)
    def _init():
        cls_acc[...] = jnp.zeros_like(cls_acc)
        np_acc[...] = jnp.zeros_like(np_acc)
        reg_ref[...] = jnp.zeros_like(reg_ref)
        cls_ref[...] = jnp.zeros_like(cls_ref)

    def lane_sum(x):
        return jnp.sum(x, axis=1, keepdims=True)

    conf = conf_ref[...]
    gt = gt_ref[...]
    pos = gt > 0
    npos = lane_sum(pos.astype(jnp.float32))
    pos_part = jnp.sum(lane_sum(jnp.where(pos, conf, 0.0)))
    k = jnp.minimum(npos * ratio, float(n))

    bits = jax.lax.bitcast_convert_type(conf, jnp.int32)
    fkey = jnp.where(bits >= 0, bits,
                     jnp.bitwise_xor(jnp.bitwise_not(bits), _I32_SIGN))
    key = jnp.where(pos, _I32_SIGN, fkey)

    t_s = jnp.full(k.shape, _I32_SIGN, jnp.int32)
    for b in range(31, -1, -1):
        mask_b = np.int32(np.uint32(1 << b))
        cand = jnp.bitwise_xor(
            jnp.bitwise_or(jnp.bitwise_xor(t_s, _I32_SIGN), mask_b),
            _I32_SIGN)
        cnt = lane_sum((key >= cand).astype(jnp.float32))
        t_s = jnp.where(cnt >= k, cand, t_s)

    above = key > t_s
    tie = key == t_s
    c_gt = lane_sum(above.astype(jnp.float32))
    r = k - c_gt

    idx = jax.lax.broadcasted_iota(jnp.int32, key.shape, 1)
    ibits = int(n).bit_length() + 1
    p = jnp.zeros(k.shape, jnp.int32)
    for b in range(ibits - 1, -1, -1):
        cand = jnp.bitwise_or(p, np.int32(1 << b))
        cnt = lane_sum((tie & (idx < cand)).astype(jnp.float32))
        p = jnp.where(cnt <= r, cand, p)

    sel = above | (tie & (idx < p))
    extra_part = jnp.sum(lane_sum(jnp.where(sel, conf, 0.0)))

    cls_acc[...] = cls_acc[...] + jnp.reshape(pos_part + extra_part, (1, 1))
    np_acc[...] = np_acc[...] + jnp.reshape(jnp.sum(npos), (1, 1))

    @pl.when(step == nsteps - 1)
    def _final():
        npf = jnp.maximum(1.0, np_acc[0, 0])
        reg_ref[...] = box_ref[...] / npf
        cls_ref[...] = cls_acc[...] / npf


def kernel(gt_bboxes, gt_labels, pred_bboxes, pred_labels):
    B, N, C = pred_labels.shape
    RATIO = 3

    R = 16 if B % 16 == 0 else B
    G = B // R
    gb = gt_bboxes.reshape(B, N * 4)
    pb = pred_bboxes.reshape(B, N * 4)

    scalar = functools.partial(pl.BlockSpec, (1, 1), lambda i: (0, 0))

    box = pl.pallas_call(
        _box_kernel,
        grid=(G,),
        in_specs=[
            pl.BlockSpec((R, N * 4), lambda i: (i, 0)),
            pl.BlockSpec((R, N * 4), lambda i: (i, 0)),
        ],
        out_specs=scalar(),
        out_shape=jax.ShapeDtypeStruct((1, 1), jnp.float32),
    )(gb, pb)

    NCHUNK = 4 if B % 4 == 0 else 1
    BC = B // NCHUNK
    gt3 = gt_labels.reshape(B, 1, N)
    conf_parts = []
    for ci in range(NCHUNK):
        pred_t = jnp.transpose(
            pred_labels[ci * BC:(ci + 1) * BC], (0, 2, 1))
        conf_parts.append(pl.pallas_call(
            _conf_kernel,
            grid=(BC,),
            in_specs=[
                pl.BlockSpec((1, C, N), lambda b: (b, 0, 0)),
                pl.BlockSpec((1, 1, N), lambda b: (b, 0, 0)),
            ],
            out_specs=pl.BlockSpec((1, 1, N), lambda b: (b, 0, 0)),
            out_shape=jax.ShapeDtypeStruct((BC, 1, N), jnp.float32),
        )(pred_t, gt3[ci * BC:(ci + 1) * BC]))
    conf = jnp.concatenate(conf_parts, axis=0).reshape(B, N)

    reg, cls, _, _ = pl.pallas_call(
        functools.partial(_loss_kernel, nsteps=G, n=N, ratio=RATIO),
        grid=(G,),
        in_specs=[
            pl.BlockSpec((R, N), lambda i: (i, 0)),
            pl.BlockSpec((R, N), lambda i: (i, 0)),
            scalar(),
        ],
        out_specs=[scalar(), scalar(), scalar(), scalar()],
        out_shape=[
            jax.ShapeDtypeStruct((1, 1), jnp.float32),
            jax.ShapeDtypeStruct((1, 1), jnp.float32),
            jax.ShapeDtypeStruct((1, 1), jnp.float32),
            jax.ShapeDtypeStruct((1, 1), jnp.float32),
        ],
    )(conf, gt_labels, box)

    return (reg[0, 0], cls[0, 0])


# speedup vs baseline: 2.7937x; 2.7937x over previous
import functools

import jax
import jax.numpy as jnp
import numpy as np
from jax.experimental import pallas as pl

_I32_SIGN = np.int32(-2147483648)


def _conf_kernel(pred_t_ref, gt_ref, conf_ref):
    x = pred_t_ref[...]
    m = jnp.max(x, axis=1, keepdims=True)
    e = jnp.exp(x - m)
    s = jnp.sum(e, axis=1, keepdims=True)
    lse = m + jnp.log(s)
    gt = gt_ref[...]
    cio = jax.lax.broadcasted_iota(jnp.int32, x.shape, 1)
    xg = jnp.sum(jnp.where(cio == gt, x, 0.0), axis=1, keepdims=True)
    conf_ref[...] = lse - xg


def _box_kernel(gb_ref, pb_ref, box_ref):
    step = pl.program_id(0)

    @pl.when(step == 0)
    def _init():
        box_ref[...] = jnp.zeros_like(box_ref)

    d = pb_ref[...] - gb_ref[...]
    ad = jnp.abs(d)
    part = jnp.sum(jnp.where(ad < 1.0, 0.5 * d * d, ad - 0.5))
    box_ref[...] = box_ref[...] + jnp.reshape(part, (1, 1))


def _loss_kernel(conf_ref, gt_ref, box_ref,
                 reg_ref, cls_ref, cls_acc, np_acc,
                 *, nsteps, n, ratio):
    step = pl.program_id(0)

    @pl.when(step == 0)
    def _init():
        cls_acc[...] = jnp.zeros_like(cls_acc)
        np_acc[...] = jnp.zeros_like(np_acc)
        reg_ref[...] = jnp.zeros_like(reg_ref)
        cls_ref[...] = jnp.zeros_like(cls_ref)

    def lane_sum(x):
        return jnp.sum(x, axis=1, keepdims=True)

    conf = conf_ref[...]
    gt = gt_ref[...]
    pos = gt > 0
    npos = lane_sum(pos.astype(jnp.float32))
    pos_part = jnp.sum(lane_sum(jnp.where(pos, conf, 0.0)))
    k = jnp.minimum(npos * ratio, float(n))

    bits = jax.lax.bitcast_convert_type(conf, jnp.int32)
    fkey = jnp.where(bits >= 0, bits,
                     jnp.bitwise_xor(jnp.bitwise_not(bits), _I32_SIGN))
    key = jnp.where(pos, _I32_SIGN, fkey)

    t_s = jnp.full(k.shape, _I32_SIGN, jnp.int32)
    for b in range(31, -1, -1):
        mask_b = np.int32(np.uint32(1 << b))
        cand = jnp.bitwise_xor(
            jnp.bitwise_or(jnp.bitwise_xor(t_s, _I32_SIGN), mask_b),
            _I32_SIGN)
        cnt = lane_sum((key >= cand).astype(jnp.float32))
        t_s = jnp.where(cnt >= k, cand, t_s)

    above = key > t_s
    tie = key == t_s
    c_gt = lane_sum(above.astype(jnp.float32))
    r = k - c_gt

    idx = jax.lax.broadcasted_iota(jnp.int32, key.shape, 1)
    ibits = int(n).bit_length() + 1
    p = jnp.zeros(k.shape, jnp.int32)
    for b in range(ibits - 1, -1, -1):
        cand = jnp.bitwise_or(p, np.int32(1 << b))
        cnt = lane_sum((tie & (idx < cand)).astype(jnp.float32))
        p = jnp.where(cnt <= r, cand, p)

    sel = above | (tie & (idx < p))
    extra_part = jnp.sum(lane_sum(jnp.where(sel, conf, 0.0)))

    cls_acc[...] = cls_acc[...] + jnp.reshape(pos_part + extra_part, (1, 1))
    np_acc[...] = np_acc[...] + jnp.reshape(jnp.sum(npos), (1, 1))

    @pl.when(step == nsteps - 1)
    def _final():
        npf = jnp.maximum(1.0, np_acc[0, 0])
        reg_ref[...] = box_ref[...] / npf
        cls_ref[...] = cls_acc[...] / npf


def kernel(gt_bboxes, gt_labels, pred_bboxes, pred_labels):
    B, N, C = pred_labels.shape
    RATIO = 3

    R = 16 if B % 16 == 0 else B
    G = B // R
    gb = gt_bboxes.reshape(B, N * 4)
    pb = pred_bboxes.reshape(B, N * 4)

    scalar = functools.partial(pl.BlockSpec, (1, 1), lambda i: (0, 0))

    box = pl.pallas_call(
        _box_kernel,
        grid=(G,),
        in_specs=[
            pl.BlockSpec((R, N * 4), lambda i: (i, 0)),
            pl.BlockSpec((R, N * 4), lambda i: (i, 0)),
        ],
        out_specs=scalar(),
        out_shape=jax.ShapeDtypeStruct((1, 1), jnp.float32),
    )(gb, pb)

    pred_t = jnp.transpose(pred_labels, (0, 2, 1))
    RB = 2 if B % 2 == 0 else 1
    conf = pl.pallas_call(
        _conf_kernel,
        grid=(B // RB,),
        in_specs=[
            pl.BlockSpec((RB, C, N), lambda b: (b, 0, 0)),
            pl.BlockSpec((RB, 1, N), lambda b: (b, 0, 0)),
        ],
        out_specs=pl.BlockSpec((RB, 1, N), lambda b: (b, 0, 0)),
        out_shape=jax.ShapeDtypeStruct((B, 1, N), jnp.float32),
    )(pred_t, gt_labels.reshape(B, 1, N)).reshape(B, N)

    reg, cls, _, _ = pl.pallas_call(
        functools.partial(_loss_kernel, nsteps=G, n=N, ratio=RATIO),
        grid=(G,),
        in_specs=[
            pl.BlockSpec((R, N), lambda i: (i, 0)),
            pl.BlockSpec((R, N), lambda i: (i, 0)),
            scalar(),
        ],
        out_specs=[scalar(), scalar(), scalar(), scalar()],
        out_shape=[
            jax.ShapeDtypeStruct((1, 1), jnp.float32),
            jax.ShapeDtypeStruct((1, 1), jnp.float32),
            jax.ShapeDtypeStruct((1, 1), jnp.float32),
            jax.ShapeDtypeStruct((1, 1), jnp.float32),
        ],
    )(conf, gt_labels, box)

    return (reg[0, 0], cls[0, 0])


# speedup vs baseline: 2.9542x; 1.0575x over previous
import functools

import jax
import jax.numpy as jnp
import numpy as np
from jax.experimental import pallas as pl

_I32_SIGN = np.int32(-2147483648)


def _conf_kernel(pred_t_ref, gt_ref, conf_ref):
    x = pred_t_ref[...]
    m = jnp.max(x, axis=1, keepdims=True)
    e = jnp.exp(x - m)
    s = jnp.sum(e, axis=1, keepdims=True)
    lse = m + jnp.log(s)
    gt = gt_ref[...]
    cio = jax.lax.broadcasted_iota(jnp.int32, x.shape, 1)
    xg = jnp.sum(jnp.where(cio == gt, x, 0.0), axis=1, keepdims=True)
    conf_ref[...] = lse - xg


def _box_kernel(gb_ref, pb_ref, box_ref):
    step = pl.program_id(0)

    @pl.when(step == 0)
    def _init():
        box_ref[...] = jnp.zeros_like(box_ref)

    d = pb_ref[...] - gb_ref[...]
    ad = jnp.abs(d)
    part = jnp.sum(jnp.where(ad < 1.0, 0.5 * d * d, ad - 0.5))
    box_ref[...] = box_ref[...] + jnp.reshape(part, (1, 1))


def _loss_kernel(conf_ref, gt_ref, box_ref,
                 reg_ref, cls_ref, cls_acc, np_acc,
                 *, nsteps, n, ratio):
    step = pl.program_id(0)

    @pl.when(step == 0)
    def _init():
        cls_acc[...] = jnp.zeros_like(cls_acc)
        np_acc[...] = jnp.zeros_like(np_acc)
        reg_ref[...] = jnp.zeros_like(reg_ref)
        cls_ref[...] = jnp.zeros_like(cls_ref)

    def lane_sum(x):
        return jnp.sum(x, axis=1, keepdims=True)

    conf = conf_ref[...]
    gt = gt_ref[...]
    pos = gt > 0
    npos = lane_sum(pos.astype(jnp.float32))
    pos_part = jnp.sum(lane_sum(jnp.where(pos, conf, 0.0)))
    k = jnp.minimum(npos * ratio, float(n))

    bits = jax.lax.bitcast_convert_type(conf, jnp.int32)
    fkey = jnp.where(bits >= 0, bits,
                     jnp.bitwise_xor(jnp.bitwise_not(bits), _I32_SIGN))
    key = jnp.where(pos, _I32_SIGN, fkey)

    t_s = jnp.full(k.shape, _I32_SIGN, jnp.int32)
    for b in range(31, -1, -1):
        mask_b = np.int32(np.uint32(1 << b))
        cand = jnp.bitwise_xor(
            jnp.bitwise_or(jnp.bitwise_xor(t_s, _I32_SIGN), mask_b),
            _I32_SIGN)
        cnt = lane_sum((key >= cand).astype(jnp.float32))
        t_s = jnp.where(cnt >= k, cand, t_s)

    above = key > t_s
    tie = key == t_s
    c_gt = lane_sum(above.astype(jnp.float32))
    r = k - c_gt

    idx = jax.lax.broadcasted_iota(jnp.int32, key.shape, 1)
    ibits = int(n).bit_length() + 1
    p = jnp.zeros(k.shape, jnp.int32)
    for b in range(ibits - 1, -1, -1):
        cand = jnp.bitwise_or(p, np.int32(1 << b))
        cnt = lane_sum((tie & (idx < cand)).astype(jnp.float32))
        p = jnp.where(cnt <= r, cand, p)

    sel = above | (tie & (idx < p))
    extra_part = jnp.sum(lane_sum(jnp.where(sel, conf, 0.0)))

    cls_acc[...] = cls_acc[...] + jnp.reshape(pos_part + extra_part, (1, 1))
    np_acc[...] = np_acc[...] + jnp.reshape(jnp.sum(npos), (1, 1))

    @pl.when(step == nsteps - 1)
    def _final():
        npf = jnp.maximum(1.0, np_acc[0, 0])
        reg_ref[...] = box_ref[...] / npf
        cls_ref[...] = cls_acc[...] / npf


def kernel(gt_bboxes, gt_labels, pred_bboxes, pred_labels):
    B, N, C = pred_labels.shape
    RATIO = 3

    R = 16 if B % 16 == 0 else B
    G = B // R
    gb = gt_bboxes.reshape(B, N * 4)
    pb = pred_bboxes.reshape(B, N * 4)

    scalar = functools.partial(pl.BlockSpec, (1, 1), lambda i: (0, 0))

    box = pl.pallas_call(
        _box_kernel,
        grid=(G,),
        in_specs=[
            pl.BlockSpec((R, N * 4), lambda i: (i, 0)),
            pl.BlockSpec((R, N * 4), lambda i: (i, 0)),
        ],
        out_specs=scalar(),
        out_shape=jax.ShapeDtypeStruct((1, 1), jnp.float32),
    )(gb, pb)

    pred_t = jnp.transpose(pred_labels, (0, 2, 1))
    RB = 4 if B % 4 == 0 else 1
    conf = pl.pallas_call(
        _conf_kernel,
        grid=(B // RB,),
        in_specs=[
            pl.BlockSpec((RB, C, N), lambda b: (b, 0, 0)),
            pl.BlockSpec((RB, 1, N), lambda b: (b, 0, 0)),
        ],
        out_specs=pl.BlockSpec((RB, 1, N), lambda b: (b, 0, 0)),
        out_shape=jax.ShapeDtypeStruct((B, 1, N), jnp.float32),
    )(pred_t, gt_labels.reshape(B, 1, N)).reshape(B, N)

    reg, cls, _, _ = pl.pallas_call(
        functools.partial(_loss_kernel, nsteps=G, n=N, ratio=RATIO),
        grid=(G,),
        in_specs=[
            pl.BlockSpec((R, N), lambda i: (i, 0)),
            pl.BlockSpec((R, N), lambda i: (i, 0)),
            scalar(),
        ],
        out_specs=[scalar(), scalar(), scalar(), scalar()],
        out_shape=[
            jax.ShapeDtypeStruct((1, 1), jnp.float32),
            jax.ShapeDtypeStruct((1, 1), jnp.float32),
            jax.ShapeDtypeStruct((1, 1), jnp.float32),
            jax.ShapeDtypeStruct((1, 1), jnp.float32),
        ],
    )(conf, gt_labels, box)

    return (reg[0, 0], cls[0, 0])


# speedup vs baseline: 3.0031x; 1.0166x over previous
import functools

import jax
import jax.numpy as jnp
import numpy as np
from jax.experimental import pallas as pl

_I32_SIGN = np.int32(-2147483648)


def _conf_kernel(pred_t_ref, gt_ref, conf_ref):
    x = pred_t_ref[...]
    m = jnp.max(x, axis=1, keepdims=True)
    e = jnp.exp(x - m)
    s = jnp.sum(e, axis=1, keepdims=True)
    lse = m + jnp.log(s)
    gt = gt_ref[...]
    cio = jax.lax.broadcasted_iota(jnp.int32, x.shape, 1)
    xg = jnp.sum(jnp.where(cio == gt, x, 0.0), axis=1, keepdims=True)
    conf_ref[...] = lse - xg


def _box_kernel(gb_ref, pb_ref, box_ref):
    step = pl.program_id(0)

    @pl.when(step == 0)
    def _init():
        box_ref[...] = jnp.zeros_like(box_ref)

    d = pb_ref[...] - gb_ref[...]
    ad = jnp.abs(d)
    part = jnp.sum(jnp.where(ad < 1.0, 0.5 * d * d, ad - 0.5))
    box_ref[...] = box_ref[...] + jnp.reshape(part, (1, 1))


def _loss_kernel(conf_ref, gt_ref, box_ref,
                 reg_ref, cls_ref, cls_acc, np_acc,
                 *, nsteps, n, ratio):
    step = pl.program_id(0)

    @pl.when(step == 0)
    def _init():
        cls_acc[...] = jnp.zeros_like(cls_acc)
        np_acc[...] = jnp.zeros_like(np_acc)
        reg_ref[...] = jnp.zeros_like(reg_ref)
        cls_ref[...] = jnp.zeros_like(cls_ref)

    def lane_sum(x):
        return jnp.sum(x, axis=1, keepdims=True)

    conf = conf_ref[...]
    gt = gt_ref[...]
    pos = gt > 0
    npos = lane_sum(pos.astype(jnp.float32))
    pos_part = jnp.sum(lane_sum(jnp.where(pos, conf, 0.0)))
    k = jnp.minimum(npos * ratio, float(n))

    bits = jax.lax.bitcast_convert_type(conf, jnp.int32)
    fkey = jnp.where(bits >= 0, bits,
                     jnp.bitwise_xor(jnp.bitwise_not(bits), _I32_SIGN))
    key = jnp.where(pos, _I32_SIGN, fkey)

    t_s = jnp.full(k.shape, _I32_SIGN, jnp.int32)
    for b in range(31, -1, -1):
        mask_b = np.int32(np.uint32(1 << b))
        cand = jnp.bitwise_xor(
            jnp.bitwise_or(jnp.bitwise_xor(t_s, _I32_SIGN), mask_b),
            _I32_SIGN)
        cnt = lane_sum((key >= cand).astype(jnp.float32))
        t_s = jnp.where(cnt >= k, cand, t_s)

    above = key > t_s
    tie = key == t_s
    c_gt = lane_sum(above.astype(jnp.float32))
    r = k - c_gt

    idx = jax.lax.broadcasted_iota(jnp.int32, key.shape, 1)
    ibits = int(n).bit_length() + 1
    p = jnp.zeros(k.shape, jnp.int32)
    for b in range(ibits - 1, -1, -1):
        cand = jnp.bitwise_or(p, np.int32(1 << b))
        cnt = lane_sum((tie & (idx < cand)).astype(jnp.float32))
        p = jnp.where(cnt <= r, cand, p)

    sel = above | (tie & (idx < p))
    extra_part = jnp.sum(lane_sum(jnp.where(sel, conf, 0.0)))

    cls_acc[...] = cls_acc[...] + jnp.reshape(pos_part + extra_part, (1, 1))
    np_acc[...] = np_acc[...] + jnp.reshape(jnp.sum(npos), (1, 1))

    @pl.when(step == nsteps - 1)
    def _final():
        npf = jnp.maximum(1.0, np_acc[0, 0])
        reg_ref[...] = box_ref[...] / npf
        cls_ref[...] = cls_acc[...] / npf


def kernel(gt_bboxes, gt_labels, pred_bboxes, pred_labels):
    B, N, C = pred_labels.shape
    RATIO = 3

    R = 16 if B % 16 == 0 else B
    G = B // R
    gb = gt_bboxes.reshape(B, N * 4)
    pb = pred_bboxes.reshape(B, N * 4)

    scalar = functools.partial(pl.BlockSpec, (1, 1), lambda i: (0, 0))

    box = pl.pallas_call(
        _box_kernel,
        grid=(G,),
        in_specs=[
            pl.BlockSpec((R, N * 4), lambda i: (i, 0)),
            pl.BlockSpec((R, N * 4), lambda i: (i, 0)),
        ],
        out_specs=scalar(),
        out_shape=jax.ShapeDtypeStruct((1, 1), jnp.float32),
    )(gb, pb)

    pred_t = jnp.transpose(pred_labels, (0, 2, 1))
    RB = 8 if B % 8 == 0 else 1
    conf = pl.pallas_call(
        _conf_kernel,
        grid=(B // RB,),
        in_specs=[
            pl.BlockSpec((RB, C, N), lambda b: (b, 0, 0)),
            pl.BlockSpec((RB, 1, N), lambda b: (b, 0, 0)),
        ],
        out_specs=pl.BlockSpec((RB, 1, N), lambda b: (b, 0, 0)),
        out_shape=jax.ShapeDtypeStruct((B, 1, N), jnp.float32),
    )(pred_t, gt_labels.reshape(B, 1, N)).reshape(B, N)

    reg, cls, _, _ = pl.pallas_call(
        functools.partial(_loss_kernel, nsteps=G, n=N, ratio=RATIO),
        grid=(G,),
        in_specs=[
            pl.BlockSpec((R, N), lambda i: (i, 0)),
            pl.BlockSpec((R, N), lambda i: (i, 0)),
            scalar(),
        ],
        out_specs=[scalar(), scalar(), scalar(), scalar()],
        out_shape=[
            jax.ShapeDtypeStruct((1, 1), jnp.float32),
            jax.ShapeDtypeStruct((1, 1), jnp.float32),
            jax.ShapeDtypeStruct((1, 1), jnp.float32),
            jax.ShapeDtypeStruct((1, 1), jnp.float32),
        ],
    )(conf, gt_labels, box)

    return (reg[0, 0], cls[0, 0])
